# baseline (speedup 1.0000x reference)
import functools

import jax
import jax.numpy as jnp
from jax.experimental import pallas as pl


def _argmin_body(res_ref, ss_ref, cb_ref, cbn_ref, idx_ref, *, K):
    res = res_ref[...]
    ss = ss_ref[...]
    cb = cb_ref[...]
    cbn = cbn_ref[0:1, :]
    bm = res.shape[0]
    mm = jax.lax.dot_general(res, cb, (((1,), (1,)), ((), ())),
                             preferred_element_type=jnp.float32)
    dist = (ss - 2.0 * mm) + cbn
    m = jnp.min(dist, axis=1, keepdims=True)
    iota = jax.lax.broadcasted_iota(jnp.int32, (bm, K), 1)
    idx_ref[...] = jnp.min(jnp.where(dist == m, iota, K), axis=1,
                           keepdims=True)


def _argmin_call(res, ss, cb, cbn8, *, bm):
    B, D = res.shape
    K = cb.shape[0]
    row = lambda i: (i, 0)
    rep = lambda i: (0, 0)
    return pl.pallas_call(
        functools.partial(_argmin_body, K=K),
        grid=(B // bm,),
        in_specs=[pl.BlockSpec((bm, D), row),
                  pl.BlockSpec((bm, 1), row),
                  pl.BlockSpec((K, D), rep),
                  pl.BlockSpec((8, K), rep)],
        out_specs=pl.BlockSpec((bm, 1), row),
        out_shape=jax.ShapeDtypeStruct((B, 1), jnp.int32),
    )(res, ss, cb, cbn8)


def _loss_body(z_ref, zq_ref, out_ref):
    i = pl.program_id(0)
    part = jnp.sum((z_ref[...] - zq_ref[...]) ** 2).reshape(1, 1)

    @pl.when(i == 0)
    def _init():
        out_ref[...] = part

    @pl.when(i != 0)
    def _acc():
        out_ref[...] += part


def _loss_call(z, zq, *, bm):
    B, D = z.shape
    row = lambda i: (i, 0)
    return pl.pallas_call(
        _loss_body,
        grid=(B // bm,),
        in_specs=[pl.BlockSpec((bm, D), row),
                  pl.BlockSpec((bm, D), row)],
        out_specs=pl.BlockSpec((1, 1), lambda i: (0, 0)),
        out_shape=jax.ShapeDtypeStruct((1, 1), jnp.float32),
    )(z, zq)


@jax.jit
def kernel(z, codebook):
    B, D = z.shape
    R, K, _ = codebook.shape
    bm = 1024
    res = z
    idxs = []
    codes = []
    for t in range(R):
        cb_t = codebook[t]
        cbn8 = jnp.broadcast_to(jnp.sum(cb_t * cb_t, axis=1)[None, :], (8, K))
        ss = jnp.sum(res * res, axis=1, keepdims=True)
        idx = _argmin_call(res, ss, cb_t, cbn8, bm=bm)
        code = jnp.take(cb_t, idx[:, 0], axis=0)
        idxs.append(idx)
        codes.append(code)
        res = res - code
    zq = jnp.sum(jnp.stack(codes, axis=0), axis=0)
    indices = jnp.concatenate(idxs, axis=1)
    loss_part = _loss_call(z, zq, bm=bm)
    commitment_loss = loss_part[0, 0] / (B * D)
    z_q_st = zq + jax.lax.stop_gradient(z - zq)
    return (z_q_st, indices, commitment_loss)


# speedup vs baseline: 1.2350x; 1.2350x over previous
import functools

import jax
import jax.numpy as jnp
from jax.experimental import pallas as pl


def _split3(cb):
    def trunc(v):
        bits = jax.lax.bitcast_convert_type(v, jnp.uint32)
        return jax.lax.bitcast_convert_type(bits & jnp.uint32(0xFFFF0000),
                                            jnp.float32)
    hi = trunc(cb)
    r1 = cb - hi
    mid = trunc(r1)
    lo = r1 - mid
    return (hi.astype(jnp.bfloat16), mid.astype(jnp.bfloat16),
            lo.astype(jnp.bfloat16))


def _stage_body(*refs, K, last):
    it = iter(refs)
    res_ref = next(it)
    ss_ref = next(it)
    cb_ref = next(it)
    chi_ref = next(it)
    cmid_ref = next(it)
    clo_ref = next(it)
    cbn_ref = next(it)
    z_ref = next(it) if last else None
    idx_ref = next(it)
    out_ref = next(it)
    loss_ref = next(it) if last else None

    res = res_ref[...]
    ss = ss_ref[...]
    cb = cb_ref[...]
    cbn = cbn_ref[0:1, :]
    bm = res.shape[0]

    mm = jax.lax.dot_general(res, cb, (((1,), (1,)), ((), ())),
                             preferred_element_type=jnp.float32)
    dist = (ss - 2.0 * mm) + cbn
    m = jnp.min(dist, axis=1, keepdims=True)
    iota = jax.lax.broadcasted_iota(jnp.int32, (bm, K), 1)
    idx = jnp.min(jnp.where(dist == m, iota, K), axis=1, keepdims=True)
    idx_ref[...] = idx

    oh = (iota == idx).astype(jnp.bfloat16)
    gat = lambda c_ref: jax.lax.dot_general(
        oh, c_ref[...], (((1,), (0,)), ((), ())),
        preferred_element_type=jnp.float32)
    code = (gat(chi_ref) + gat(cmid_ref)) + gat(clo_ref)
    new_res = res - code

    if not last:
        out_ref[...] = new_res
    else:
        zb = z_ref[...]
        zq = zb - new_res
        out_ref[...] = zq
        part = jnp.sum((zb - zq) ** 2).reshape(1, 1)
        i = pl.program_id(0)

        @pl.when(i == 0)
        def _init():
            loss_ref[...] = part

        @pl.when(i != 0)
        def _acc():
            loss_ref[...] += part


def _stage_call(res, ss, cb, splits, cbn8, z, *, last, bm):
    B, D = res.shape
    K = cb.shape[0]
    row = lambda i: (i, 0)
    rep = lambda i: (0, 0)
    in_specs = [pl.BlockSpec((bm, D), row),
                pl.BlockSpec((bm, 1), row),
                pl.BlockSpec((K, D), rep),
                pl.BlockSpec((K, D), rep),
                pl.BlockSpec((K, D), rep),
                pl.BlockSpec((K, D), rep),
                pl.BlockSpec((8, K), rep)]
    args = [res, ss, cb, *splits, cbn8]
    if last:
        in_specs.append(pl.BlockSpec((bm, D), row))
        args.append(z)
    out_specs = [pl.BlockSpec((bm, 1), row),
                 pl.BlockSpec((bm, D), row)]
    out_shape = [jax.ShapeDtypeStruct((B, 1), jnp.int32),
                 jax.ShapeDtypeStruct((B, D), jnp.float32)]
    if last:
        out_specs.append(pl.BlockSpec((1, 1), rep))
        out_shape.append(jax.ShapeDtypeStruct((1, 1), jnp.float32))
    return pl.pallas_call(
        functools.partial(_stage_body, K=K, last=last),
        grid=(B // bm,),
        in_specs=in_specs,
        out_specs=out_specs,
        out_shape=out_shape,
    )(*args)


@jax.jit
def kernel(z, codebook):
    B, D = z.shape
    R, K, _ = codebook.shape
    bm = 1024
    res = z
    idxs = []
    for t in range(R):
        cb_t = codebook[t]
        cbn8 = jnp.broadcast_to(jnp.sum(cb_t * cb_t, axis=1)[None, :], (8, K))
        ss = jnp.sum(res * res, axis=1, keepdims=True)
        last = t == R - 1
        outs = _stage_call(res, ss, cb_t, _split3(cb_t), cbn8,
                           z if last else None, last=last, bm=bm)
        if last:
            idx, zq, loss_part = outs
        else:
            idx, res = outs
        idxs.append(idx)
    indices = jnp.concatenate(idxs, axis=1)
    commitment_loss = loss_part[0, 0] / (B * D)
    z_q_st = zq + jax.lax.stop_gradient(z - zq)
    return (z_q_st, indices, commitment_loss)


# speedup vs baseline: 1.5886x; 1.2863x over previous
import functools

import jax
import jax.numpy as jnp
from jax.experimental import pallas as pl


def _split3(cb):
    def trunc(v):
        bits = jax.lax.bitcast_convert_type(v, jnp.uint32)
        return jax.lax.bitcast_convert_type(bits & jnp.uint32(0xFFFF0000),
                                            jnp.float32)
    hi = trunc(cb)
    r1 = cb - hi
    mid = trunc(r1)
    lo = r1 - mid
    return (hi.astype(jnp.bfloat16), mid.astype(jnp.bfloat16),
            lo.astype(jnp.bfloat16))


def _stage_body(*refs, K, last):
    it = iter(refs)
    res_ref = next(it)
    ss_ref = next(it)
    cb_ref = next(it)
    cbr_ref = next(it)
    cbn_ref = next(it)
    z_ref = next(it) if last else None
    idx_ref = next(it)
    out_ref = next(it)
    loss_ref = next(it) if last else None

    res = res_ref[...]
    ss = ss_ref[...]
    cb = cb_ref[...]
    cbn = cbn_ref[0:1, :]
    bm, D = res.shape
    NH = K // 256

    mm = jax.lax.dot_general(res, cb, (((1,), (1,)), ((), ())),
                             preferred_element_type=jnp.float32)
    dist = (ss - 2.0 * mm) + cbn
    m = jnp.min(dist, axis=1, keepdims=True)
    iota = jax.lax.broadcasted_iota(jnp.int32, (bm, K), 1)
    idx = jnp.min(jnp.where(dist == m, iota, K), axis=1, keepdims=True)
    idx_ref[...] = idx

    lo = idx & 255
    hi = idx >> 8
    iota768 = jax.lax.broadcasted_iota(jnp.int32, (bm, 768), 1)
    ohl3 = ((iota768 & 255) == lo).astype(jnp.bfloat16)
    part = jax.lax.dot_general(ohl3, cbr_ref[...], (((1,), (0,)), ((), ())),
                               preferred_element_type=jnp.float32)
    code = None
    for h in range(NH):
        sel = (hi == h).astype(jnp.float32)
        chunk = sel * part[:, h * D:(h + 1) * D]
        code = chunk if code is None else code + chunk
    new_res = res - code

    if not last:
        out_ref[...] = new_res
    else:
        zb = z_ref[...]
        zq = zb - new_res
        out_ref[...] = zq
        part = jnp.sum((zb - zq) ** 2).reshape(1, 1)
        i = pl.program_id(0)

        @pl.when(i == 0)
        def _init():
            loss_ref[...] = part

        @pl.when(i != 0)
        def _acc():
            loss_ref[...] += part


def _stage_call(res, ss, cb, cbr, cbn8, z, *, last, bm):
    B, D = res.shape
    K = cb.shape[0]
    row = lambda i: (i, 0)
    rep = lambda i: (0, 0)
    in_specs = [pl.BlockSpec((bm, D), row),
                pl.BlockSpec((bm, 1), row),
                pl.BlockSpec((K, D), rep),
                pl.BlockSpec(cbr.shape, rep),
                pl.BlockSpec((8, K), rep)]
    args = [res, ss, cb, cbr, cbn8]
    if last:
        in_specs.append(pl.BlockSpec((bm, D), row))
        args.append(z)
    out_specs = [pl.BlockSpec((bm, 1), row),
                 pl.BlockSpec((bm, D), row)]
    out_shape = [jax.ShapeDtypeStruct((B, 1), jnp.int32),
                 jax.ShapeDtypeStruct((B, D), jnp.float32)]
    if last:
        out_specs.append(pl.BlockSpec((1, 1), rep))
        out_shape.append(jax.ShapeDtypeStruct((1, 1), jnp.float32))
    return pl.pallas_call(
        functools.partial(_stage_body, K=K, last=last),
        grid=(B // bm,),
        in_specs=in_specs,
        out_specs=out_specs,
        out_shape=out_shape,
    )(*args)


@jax.jit
def kernel(z, codebook):
    B, D = z.shape
    R, K, _ = codebook.shape
    bm = 2048
    NH = K // 256
    res = z
    idxs = []
    for t in range(R):
        cb_t = codebook[t]
        cbn8 = jnp.broadcast_to(jnp.sum(cb_t * cb_t, axis=1)[None, :], (8, K))
        cbr = jnp.concatenate(
            [p.reshape(NH, 256, D).transpose(1, 0, 2).reshape(256, NH * D)
             for p in _split3(cb_t)], axis=0)
        ss = jnp.sum(res * res, axis=1, keepdims=True)
        last = t == R - 1
        outs = _stage_call(res, ss, cb_t, cbr, cbn8,
                           z if last else None, last=last, bm=bm)
        if last:
            idx, zq, loss_part = outs
        else:
            idx, res = outs
        idxs.append(idx)
    indices = jnp.concatenate(idxs, axis=1)
    commitment_loss = loss_part[0, 0] / (B * D)
    z_q_st = zq + jax.lax.stop_gradient(z - zq)
    return (z_q_st, indices, commitment_loss)


# speedup vs baseline: 1.7604x; 1.1081x over previous
import functools

import jax
import jax.numpy as jnp
from jax.experimental import pallas as pl


def _split3(cb):
    def trunc(v):
        bits = jax.lax.bitcast_convert_type(v, jnp.uint32)
        return jax.lax.bitcast_convert_type(bits & jnp.uint32(0xFFFF0000),
                                            jnp.float32)
    hi = trunc(cb)
    r1 = cb - hi
    mid = trunc(r1)
    lo = r1 - mid
    return (hi.astype(jnp.bfloat16), mid.astype(jnp.bfloat16),
            lo.astype(jnp.bfloat16))


def _stage_body(*refs, K, last, nblk, scale):
    it = iter(refs)
    res_ref = next(it)
    ss_ref = next(it)
    cb_ref = next(it)
    cbr_ref = next(it)
    cbn_ref = next(it)
    z_ref = next(it) if last else None
    idx_ref = next(it)
    out_ref = next(it)
    loss_ref = next(it) if last else None

    res = res_ref[...]
    ss = ss_ref[...]
    cb = cb_ref[0]
    cbn = cbn_ref[0]
    bm, D = res.shape

    mm = jax.lax.dot_general(res, cb, (((1,), (1,)), ((), ())),
                             preferred_element_type=jnp.float32)
    dist = (ss - 2.0 * mm) + cbn
    m = jnp.min(dist, axis=1, keepdims=True)
    iota = jax.lax.broadcasted_iota(jnp.int32, (bm, K), 1)
    idx = jnp.min(jnp.where(dist == m, iota, K), axis=1, keepdims=True)
    idx_ref[...] = idx

    grp = idx >> 2
    sub = idx & 3
    iota768 = jax.lax.broadcasted_iota(jnp.int32, (bm, 768), 1)
    ohl3 = ((iota768 & 255) == grp).astype(jnp.bfloat16)
    part = jax.lax.dot_general(ohl3, cbr_ref[0], (((1,), (0,)), ((), ())),
                               preferred_element_type=jnp.float32)
    code = None
    for h in range(4):
        sel = (sub == h).astype(jnp.float32)
        chunk = sel * part[:, h * D:(h + 1) * D]
        code = chunk if code is None else code + chunk
    new_res = res - code

    if not last:
        out_ref[...] = new_res
    else:
        zb = z_ref[...]
        zq = zb - new_res
        out_ref[...] = zq + (zb - zq)
        part_l = jnp.sum((zb - zq) ** 2).reshape(1, 1)
        i = pl.program_id(0)

        @pl.when(i == 0)
        def _init():
            loss_ref[...] = part_l

        @pl.when(jnp.logical_and(i != 0, i != nblk - 1))
        def _acc():
            loss_ref[...] += part_l

        @pl.when(jnp.logical_and(i != 0, i == nblk - 1))
        def _fin():
            loss_ref[...] = (loss_ref[...] + part_l) * scale


def _stage_call(res, ss, codebook, cbr, cbn, z, *, t, last, bm):
    B, D = res.shape
    R, K, _ = codebook.shape
    nblk = B // bm
    row = lambda i: (i, 0)
    in_specs = [pl.BlockSpec((bm, D), row),
                pl.BlockSpec((bm, 1), row),
                pl.BlockSpec((1, K, D), lambda i: (t, 0, 0)),
                pl.BlockSpec((1, 768, 256), lambda i: (t, 0, 0)),
                pl.BlockSpec((1, 1, K), lambda i: (t, 0, 0))]
    args = [res, ss, codebook, cbr, cbn]
    if last:
        in_specs.append(pl.BlockSpec((bm, D), row))
        args.append(z)
    out_specs = [pl.BlockSpec((bm, 1), row),
                 pl.BlockSpec((bm, D), row)]
    out_shape = [jax.ShapeDtypeStruct((B, 1), jnp.int32),
                 jax.ShapeDtypeStruct((B, D), jnp.float32)]
    if last:
        out_specs.append(pl.BlockSpec((1, 1), lambda i: (0, 0)))
        out_shape.append(jax.ShapeDtypeStruct((1, 1), jnp.float32))
    return pl.pallas_call(
        functools.partial(_stage_body, K=K, last=last, nblk=nblk,
                          scale=1.0 / (B * D)),
        grid=(nblk,),
        in_specs=in_specs,
        out_specs=out_specs,
        out_shape=out_shape,
    )(*args)


@jax.jit
def kernel(z, codebook):
    B, D = z.shape
    R, K, _ = codebook.shape
    bm = 2048

    cbn = jnp.sum(codebook * codebook, axis=2)[:, None, :]
    planes = _split3(codebook)
    cbr = jnp.concatenate([p.reshape(R, K // 4, 4 * D) for p in planes],
                          axis=1)

    res = z
    idxs = []
    for t in range(R):
        ss = jnp.sum(res * res, axis=1, keepdims=True)
        last = t == R - 1
        outs = _stage_call(res, ss, codebook, cbr, cbn,
                           z if last else None, t=t, last=last, bm=bm)
        if last:
            idx, z_q_st, loss_part = outs
        else:
            idx, res = outs
        idxs.append(idx)
    indices = jnp.concatenate(idxs, axis=1)
    return (z_q_st, indices, loss_part[0, 0])
